# baseline (speedup 1.0000x reference)
import jax
import jax.numpy as jnp
from jax.experimental import pallas as pl
from jax.experimental.pallas import tpu as pltpu


def _copy_body(x_ref, e_ref, x_out, e_out, x_sem, e_sem):
    cx = pltpu.make_async_copy(x_ref, x_out, x_sem)
    ce = pltpu.make_async_copy(e_ref, e_out, e_sem)
    cx.start()
    ce.start()
    cx.wait()
    ce.wait()


def kernel(x, x_lstm, encoded_z_gnss, edge_index, edge_attr):
    x_out, e_out = pl.pallas_call(
        _copy_body,
        out_shape=(
            jax.ShapeDtypeStruct(x.shape, x.dtype),
            jax.ShapeDtypeStruct(edge_attr.shape, edge_attr.dtype),
        ),
        in_specs=[
            pl.BlockSpec(memory_space=pl.ANY),
            pl.BlockSpec(memory_space=pl.ANY),
        ],
        out_specs=(
            pl.BlockSpec(memory_space=pl.ANY),
            pl.BlockSpec(memory_space=pl.ANY),
        ),
        scratch_shapes=[pltpu.SemaphoreType.DMA, pltpu.SemaphoreType.DMA],
    )(x, edge_attr)
    return (x_out, e_out)


# speedup vs baseline: 20.1583x; 20.1583x over previous
import jax
import jax.numpy as jnp
from jax.experimental import pallas as pl
from jax.experimental.pallas import tpu as pltpu

_GRID = 10


def _copy_body(x_ref, e_ref, x_out, e_out):
    x_out[...] = x_ref[...]
    e_out[...] = e_ref[...]


def kernel(x, x_lstm, encoded_z_gnss, edge_index, edge_attr):
    n_nodes, d_feat = x.shape
    n_edges, d_edge = edge_attr.shape
    bx = n_nodes // _GRID
    be = n_edges // _GRID
    x_out, e_out = pl.pallas_call(
        _copy_body,
        grid=(_GRID,),
        out_shape=(
            jax.ShapeDtypeStruct(x.shape, x.dtype),
            jax.ShapeDtypeStruct(edge_attr.shape, edge_attr.dtype),
        ),
        in_specs=[
            pl.BlockSpec((bx, d_feat), lambda i: (i, 0)),
            pl.BlockSpec((be, d_edge), lambda i: (i, 0)),
        ],
        out_specs=(
            pl.BlockSpec((bx, d_feat), lambda i: (i, 0)),
            pl.BlockSpec((be, d_edge), lambda i: (i, 0)),
        ),
        compiler_params=pltpu.CompilerParams(
            dimension_semantics=("arbitrary",),
        ),
    )(x, edge_attr)
    return (x_out, e_out)
